# baseline (speedup 1.0000x reference)
import functools

import jax
import jax.numpy as jnp
from jax import lax
from jax.experimental import pallas as pl
from jax.experimental.pallas import tpu as pltpu
from jax.experimental.pallas import tpu_sc as plsc

P = 2
NE = 1024
ED = 256
BETA = 0.25

M = 512


def _vq_tc_kernel(x_ref, wpe_ref, bpe_ref, g_ref, b_ref, embT_ref, emb_ref,
                  wpu_ref, bpu_ref, idx_ref, loss_ref, embp_ref):
    i = pl.program_id(0)
    n = pl.num_programs(0)
    x = x_ref[...]
    zp = jnp.dot(x, wpe_ref[...], preferred_element_type=jnp.float32) + bpe_ref[...]
    mu = jnp.mean(zp, axis=1, keepdims=True)
    zc = zp - mu
    var = jnp.mean(zc * zc, axis=1, keepdims=True)
    zp = zc / jnp.sqrt(var + 1e-5) * g_ref[...] + b_ref[...]

    emb = emb_ref[...]
    esq = jnp.sum(emb * emb, axis=1)[None, :]
    rsq = jnp.sum(zp * zp, axis=1, keepdims=True)
    scores = jnp.dot(zp, embT_ref[...], preferred_element_type=jnp.float32)
    dist = rsq + esq - 2.0 * scores
    minv = jnp.min(dist, axis=1, keepdims=True)
    cols = jax.lax.broadcasted_iota(jnp.int32, dist.shape, 1)
    idx = jnp.min(jnp.where(dist == minv, cols, NE), axis=1)
    idx_ref[0, 0, :] = idx

    s2 = jnp.sum(minv).reshape(1, 1)

    @pl.when(i == 0)
    def _():
        loss_ref[...] = s2
        embp_ref[...] = (jnp.dot(emb, wpu_ref[...], preferred_element_type=jnp.float32)
                         + bpu_ref[...])

    @pl.when(i != 0)
    def _():
        loss_ref[...] = loss_ref[...] + s2

    @pl.when(i == n - 1)
    def _():
        loss_ref[...] = loss_ref[...] * ((1.0 + BETA) / (n * M * ED))


def _make_sc_gather(B, D, NC, NS):
    NW = NC * NS
    bw = B // NW
    CH = 112
    nch = bw // CH
    mesh = plsc.VectorSubcoreMesh(core_axis_name="c", subcore_axis_name="s")

    @functools.partial(
        pl.kernel, mesh=mesh,
        out_type=jax.ShapeDtypeStruct((B, D), jnp.float32),
        scratch_types=[
            pltpu.VMEM((bw,), jnp.int32),
            pltpu.VMEM((CH, D), jnp.float32),
            pltpu.SemaphoreType.DMA,
        ],
    )
    def gather(table_hbm, idx_hbm, out_hbm, idx_v, rows_v, sem):
        wid = lax.axis_index("s") * NC + lax.axis_index("c")
        base = wid * bw
        pltpu.sync_copy(idx_hbm.at[pl.ds(base, bw)], idx_v)
        for c in range(nch):
            pltpu.async_copy(table_hbm.at[idx_v.at[pl.ds(c * CH, CH)]], rows_v, sem).wait()
            pltpu.sync_copy(rows_v, out_hbm.at[pl.ds(base + c * CH, CH)])

    return gather


def kernel(z, emb, W_pe, b_pe, gamma, beta_ln, W_pu, b_pu):
    b, c, h, w = z.shape
    hp, wp = h // P, w // P
    D = c * P * P
    patches = z.reshape(b, c, hp, P, wp, P).transpose(0, 2, 4, 1, 3, 5).reshape(b * hp * wp, D)
    N = patches.shape[0]
    grid = N // M

    idx3, loss, embP = pl.pallas_call(
        _vq_tc_kernel,
        grid=(grid,),
        in_specs=[
            pl.BlockSpec((M, D), lambda i: (i, 0)),
            pl.BlockSpec((D, ED), lambda i: (0, 0)),
            pl.BlockSpec((1, ED), lambda i: (0, 0)),
            pl.BlockSpec((1, ED), lambda i: (0, 0)),
            pl.BlockSpec((1, ED), lambda i: (0, 0)),
            pl.BlockSpec((ED, NE), lambda i: (0, 0)),
            pl.BlockSpec((NE, ED), lambda i: (0, 0)),
            pl.BlockSpec((ED, D), lambda i: (0, 0)),
            pl.BlockSpec((1, D), lambda i: (0, 0)),
        ],
        out_specs=[
            pl.BlockSpec((1, 1, M), lambda i: (i, 0, 0)),
            pl.BlockSpec((1, 1), lambda i: (0, 0)),
            pl.BlockSpec((NE, D), lambda i: (0, 0)),
        ],
        out_shape=[
            jax.ShapeDtypeStruct((grid, 1, M), jnp.int32),
            jax.ShapeDtypeStruct((1, 1), jnp.float32),
            jax.ShapeDtypeStruct((NE, D), jnp.float32),
        ],
    )(patches, W_pe.T, b_pe.reshape(1, ED), gamma.reshape(1, ED),
      beta_ln.reshape(1, ED), emb.T, emb, W_pu.T, b_pu.reshape(1, D))

    idx = idx3.reshape(N)
    info = plsc.get_sparse_core_info()
    out_p = _make_sc_gather(N, D, info.num_cores, info.num_subcores)(embP, idx)
    out = out_p.reshape(b, hp, wp, c, P, P).transpose(0, 3, 1, 4, 2, 5).reshape(b, c, h, w)
    return out, loss[0, 0], idx
